# baseline (speedup 1.0000x reference)
import functools

import jax
import jax.numpy as jnp
from jax import lax
from jax.experimental import pallas as pl
from jax.experimental.pallas import tpu as pltpu
from jax.experimental.pallas import tpu_sc as plsc

N = 8192
HID = 64
KNN = 20
NSEG = 8
NCOMP = 512
CH = 512
NCHUNK = N // CH
RT = 128
RE = 128
RP = 1024
KPAD = 32
NEG_BIG = -3.0e38


def _elu(v):
    return jnp.where(v > 0, v, jnp.exp(jnp.minimum(v, 0.0)) - 1.0)


def _ffm_body(x_ref, w_ref, b_ref, o_ref):
    o_ref[...] = jnp.dot(x_ref[...], w_ref[...],
                         preferred_element_type=jnp.float32) + b_ref[...]


def _ffm(xp, wp, b):
    return pl.pallas_call(
        _ffm_body,
        grid=(N // RP,),
        in_specs=[pl.BlockSpec((RP, 8), lambda i: (i, 0)),
                  pl.BlockSpec((8, HID), lambda i: (0, 0)),
                  pl.BlockSpec((1, HID), lambda i: (0, 0))],
        out_specs=pl.BlockSpec((RP, HID), lambda i: (i, 0)),
        out_shape=jax.ShapeDtypeStruct((N, HID), jnp.float32),
    )(xp, wp, b)


def _pre_body(h_ref, w1_ref, b1_ref, a_ref, bmat_ref, sq_ref):
    h = h_ref[...]
    w1a = w1_ref[:HID, :]
    w1b = w1_ref[HID:, :]
    a_ref[...] = jnp.dot(h, w1a - w1b,
                         preferred_element_type=jnp.float32) + b1_ref[...]
    bmat_ref[...] = jnp.dot(h, w1b, preferred_element_type=jnp.float32)
    sq_ref[...] = jnp.sum(h * h, axis=1, keepdims=True)


def _pre(h, w1, b1):
    return pl.pallas_call(
        _pre_body,
        grid=(N // RP,),
        in_specs=[pl.BlockSpec((RP, HID), lambda i: (i, 0)),
                  pl.BlockSpec((2 * HID, HID), lambda i: (0, 0)),
                  pl.BlockSpec((1, HID), lambda i: (0, 0))],
        out_specs=[pl.BlockSpec((RP, HID), lambda i: (i, 0)),
                   pl.BlockSpec((RP, HID), lambda i: (i, 0)),
                   pl.BlockSpec((RP, 1), lambda i: (i, 0))],
        out_shape=[jax.ShapeDtypeStruct((N, HID), jnp.float32),
                   jax.ShapeDtypeStruct((N, HID), jnp.float32),
                   jax.ShapeDtypeStruct((N, 1), jnp.float32)],
    )(h, w1, b1)


def _topk_body(bounds_ref, hr_ref, hall_ref, sqr_ref, sqc_ref, br_ref,
               bc_ref, idx_ref, d_ref):
    i = pl.program_id(0)
    c0 = bounds_ref[i, 0]
    c1 = bounds_ref[i, 1]
    hr = hr_ref[0]
    sqr = sqr_ref[0]
    br = br_ref[0]

    def fill(c, _):
        hc = hall_ref[c]
        dot = lax.dot_general(hc, hr, (((1,), (1,)), ((), ())),
                              preferred_element_type=jnp.float32)
        d = (sqr - 2.0 * dot) + sqc_ref[c]
        d = jnp.where(br != bc_ref[c], jnp.inf, d)
        d_ref[c] = d
        return 0

    lax.fori_loop(c0, c1, fill, 0)

    col = lax.broadcasted_iota(jnp.int32, (CH, RT), 0)
    big = jnp.int32(2 ** 30)
    prev = jnp.full((1, RT), -1, jnp.int32)
    picks = []
    for _ in range(KNN):
        def scan(c, carry):
            m, mi = carry
            v = d_ref[c]
            colg = col + c * CH
            v = jnp.where(colg == prev, jnp.inf, v)
            d_ref[c] = v
            mc = jnp.min(v, axis=0, keepdims=True)
            ic = jnp.min(jnp.where(v == mc, colg, big), axis=0,
                         keepdims=True)
            upd = (mc < m) | ((mc == m) & (ic < mi))
            return jnp.where(upd, mc, m), jnp.where(upd, ic, mi)

        _, mi = lax.fori_loop(
            c0, c1, scan,
            (jnp.full((1, RT), jnp.inf, jnp.float32),
             jnp.full((1, RT), big, jnp.int32)))
        mi = jnp.clip(mi, 0, N - 1)
        picks.append(mi)
        prev = mi
    pad = [picks[-1]] * (KPAD - len(picks))
    idx_ref[0] = jnp.concatenate(picks + pad, axis=0)


def _topk(h3, sqrT, sqc, brT, bc, bounds):
    nb = N // RT
    grid_spec = pltpu.PrefetchScalarGridSpec(
        num_scalar_prefetch=1,
        grid=(nb,),
        in_specs=[pl.BlockSpec((1, RT, HID), lambda i, s: (i, 0, 0)),
                  pl.BlockSpec((NCHUNK, CH, HID), lambda i, s: (0, 0, 0)),
                  pl.BlockSpec((1, 1, RT), lambda i, s: (i, 0, 0)),
                  pl.BlockSpec((NCHUNK, CH, 1), lambda i, s: (0, 0, 0)),
                  pl.BlockSpec((1, 1, RT), lambda i, s: (i, 0, 0)),
                  pl.BlockSpec((NCHUNK, CH, 1), lambda i, s: (0, 0, 0))],
        out_specs=pl.BlockSpec((1, KPAD, RT), lambda i, s: (i, 0, 0)),
        scratch_shapes=[pltpu.VMEM((NCHUNK, CH, RT), jnp.float32)],
    )
    hrows = h3.reshape(nb, RT, HID)
    return pl.pallas_call(
        _topk_body,
        grid_spec=grid_spec,
        out_shape=jax.ShapeDtypeStruct((nb, KPAD, RT), jnp.int32),
    )(bounds, hrows, h3, sqrT, sqc, brT, bc)


def _topk_fixed(h, sq, bf32, bounds):
    nb = N // RT
    return _topk(h.reshape(NCHUNK, CH, HID),
                 sq.reshape(nb, 1, RT),
                 sq.reshape(NCHUNK, CH, 1),
                 bf32.reshape(nb, 1, RT),
                 bf32.reshape(NCHUNK, CH, 1),
                 bounds)


def _make_gather():
    info = plsc.get_sparse_core_info()
    nw = info.num_cores * info.num_subcores
    total = N * KNN
    bpw = total // nw
    chunk = 512
    nchunks = bpw // chunk
    mesh = plsc.VectorSubcoreMesh(core_axis_name="c", subcore_axis_name="s")

    @functools.partial(
        pl.kernel, mesh=mesh,
        out_type=jax.ShapeDtypeStruct((total, HID), jnp.float32),
        scratch_types=[pltpu.VMEM((bpw,), jnp.int32),
                       pltpu.VMEM((chunk, HID), jnp.float32),
                       pltpu.SemaphoreType.DMA],
        compiler_params=pltpu.CompilerParams(use_tc_tiling_on_sc=False),
    )
    def gather(table_hbm, idx_hbm, out_hbm, idx_v, rows_v, sem):
        wid = lax.axis_index("s") * info.num_cores + lax.axis_index("c")
        base = wid * bpw
        pltpu.sync_copy(idx_hbm.at[pl.ds(base, bpw)], idx_v)
        for t in range(nchunks):
            idx_slice = idx_v.at[pl.ds(t * chunk, chunk)]
            pltpu.async_copy(table_hbm.at[idx_slice], rows_v, sem).wait()
            pltpu.sync_copy(rows_v,
                            out_hbm.at[pl.ds(base + t * chunk, chunk)])

    return gather


def _edge_body(a_ref, bg_ref, h_ref, w2_ref, b2_ref, w3_ref, b3_ref,
               g_ref, bl_ref, wl_ref, bll_ref, rc_ref, o_ref):
    a = a_ref[...]
    ae = jnp.broadcast_to(a[None, :, :], (KNN, RE, HID)).reshape(
        KNN * RE, HID)
    m1 = _elu(ae + bg_ref[...])
    m2 = _elu(jnp.dot(m1, w2_ref[...],
                      preferred_element_type=jnp.float32) + b2_ref[...])
    m3 = jnp.dot(m2, w3_ref[...],
                 preferred_element_type=jnp.float32) + b3_ref[...]
    agg = jnp.max(m3.reshape(KNN, RE, HID), axis=0)
    mu = jnp.mean(agg, axis=1, keepdims=True)
    var = jnp.mean((agg - mu) ** 2, axis=1, keepdims=True)
    y = (agg - mu) / jnp.sqrt(var + 1e-5) * g_ref[...] + bl_ref[...]
    o = jnp.dot(_elu(y), wl_ref[...],
                preferred_element_type=jnp.float32) + bll_ref[...]
    o_ref[...] = h_ref[...] + rc_ref[0, 0] * o


def _edge(a, bg, h, w2, b2, w3, b3, g, bln, wl, bl, rc):
    return pl.pallas_call(
        _edge_body,
        grid=(N // RE,),
        in_specs=[pl.BlockSpec((RE, HID), lambda i: (i, 0)),
                  pl.BlockSpec((RE * KNN, HID), lambda i: (i, 0)),
                  pl.BlockSpec((RE, HID), lambda i: (i, 0)),
                  pl.BlockSpec((HID, HID), lambda i: (0, 0)),
                  pl.BlockSpec((1, HID), lambda i: (0, 0)),
                  pl.BlockSpec((HID, HID), lambda i: (0, 0)),
                  pl.BlockSpec((1, HID), lambda i: (0, 0)),
                  pl.BlockSpec((1, HID), lambda i: (0, 0)),
                  pl.BlockSpec((1, HID), lambda i: (0, 0)),
                  pl.BlockSpec((HID, HID), lambda i: (0, 0)),
                  pl.BlockSpec((1, HID), lambda i: (0, 0)),
                  pl.BlockSpec((1, 1), lambda i: (0, 0))],
        out_specs=pl.BlockSpec((RE, HID), lambda i: (i, 0)),
        out_shape=jax.ShapeDtypeStruct((N, HID), jnp.float32),
    )(a, bg, h, w2, b2, w3, b3, g, bln, wl, bl, rc)


def _reg_body(h_ref, br_ref, w0_ref, b0_ref, w1_ref, b1_ref, w2_ref,
              b2_ref, w3_ref, b3_ref, o_ref, pool_ref):
    i = pl.program_id(0)
    nb = pl.num_programs(0)

    @pl.when(i == 0)
    def _():
        pool_ref[...] = jnp.full((NSEG, HID), NEG_BIG, jnp.float32)

    r = jnp.dot(h_ref[...], w0_ref[...],
                preferred_element_type=jnp.float32) + b0_ref[...]
    br = br_ref[...]
    for s in range(NSEG):
        rs = jnp.where(br == jnp.float32(s), r, NEG_BIG)
        ps = jnp.max(rs, axis=0, keepdims=True)
        pool_ref[pl.ds(s, 1), :] = jnp.maximum(pool_ref[pl.ds(s, 1), :], ps)

    @pl.when(i == nb - 1)
    def _():
        p = pool_ref[...]
        t = _elu(jnp.dot(p, w1_ref[...],
                         preferred_element_type=jnp.float32) + b1_ref[...])
        t = _elu(jnp.dot(t, w2_ref[...],
                         preferred_element_type=jnp.float32) + b2_ref[...])
        o_ref[...] = jnp.dot(t, w3_ref[...],
                             preferred_element_type=jnp.float32) + b3_ref[...]


def _reg(h, br, ws):
    (w0, b0), (w1, b1), (w2, b2), (w3, b3) = ws
    return pl.pallas_call(
        _reg_body,
        grid=(N // RP,),
        in_specs=[pl.BlockSpec((RP, HID), lambda i: (i, 0)),
                  pl.BlockSpec((RP, 1), lambda i: (i, 0)),
                  pl.BlockSpec((HID, HID), lambda i: (0, 0)),
                  pl.BlockSpec((1, HID), lambda i: (0, 0)),
                  pl.BlockSpec((HID, HID), lambda i: (0, 0)),
                  pl.BlockSpec((1, HID), lambda i: (0, 0)),
                  pl.BlockSpec((HID, HID), lambda i: (0, 0)),
                  pl.BlockSpec((1, HID), lambda i: (0, 0)),
                  pl.BlockSpec((HID, NCOMP), lambda i: (0, 0)),
                  pl.BlockSpec((1, NCOMP), lambda i: (0, 0))],
        out_specs=pl.BlockSpec((NSEG, NCOMP), lambda i: (0, 0)),
        out_shape=jax.ShapeDtypeStruct((NSEG, NCOMP), jnp.float32),
        scratch_shapes=[pltpu.VMEM((NSEG, HID), jnp.float32)],
    )(h, br, w0, b0, w1, b1, w2, b2, w3, b3)


def kernel(x, batch, params):
    xp = jnp.pad(x, ((0, 0), (0, 5)))
    wf = jnp.pad(params['ffm'][0], ((0, 5), (0, 0)))
    bf = params['ffm'][1].reshape(1, HID)
    h = _ffm(xp, wf, bf)

    batch = batch.astype(jnp.int32)
    bf32 = batch.astype(jnp.float32)
    br_rows = bf32.reshape(N, 1)

    seg_ids = jnp.arange(NSEG, dtype=batch.dtype)
    seg_start = jnp.searchsorted(batch, seg_ids, side='left')
    seg_end = jnp.searchsorted(batch, seg_ids, side='right')
    b_lo = batch[::RT]
    b_hi = batch[RT - 1::RT]
    c0 = (seg_start[b_lo] // CH).astype(jnp.int32)
    c1 = ((seg_end[b_hi] + CH - 1) // CH).astype(jnp.int32)
    bounds = jnp.stack([c0, c1], axis=1)

    gather = _make_gather()

    for lp in params['layers']:
        a, bmat, sq = _pre(h, lp['mlp1'][0], lp['mlp1'][1].reshape(1, HID))
        idx = _topk_fixed(h, sq, bf32, bounds)
        idxf = idx[:, :KNN, :].reshape(N * KNN)
        bg = gather(bmat, idxf)
        h = _edge(a, bg, h,
                  lp['mlp2'][0], lp['mlp2'][1].reshape(1, HID),
                  lp['mlp3'][0], lp['mlp3'][1].reshape(1, HID),
                  lp['ln_g'].reshape(1, HID), lp['ln_b'].reshape(1, HID),
                  lp['lin'][0], lp['lin'][1].reshape(1, HID),
                  lp['rc'].reshape(1, 1))

    ws = [(params['reg'][j][0],
           params['reg'][j][1].reshape(1, -1)) for j in range(4)]
    return _reg(h, br_rows, ws)


# speedup vs baseline: 1.0751x; 1.0751x over previous
import functools

import jax
import jax.numpy as jnp
from jax import lax
from jax.experimental import pallas as pl
from jax.experimental.pallas import tpu as pltpu
from jax.experimental.pallas import tpu_sc as plsc

N = 8192
HID = 64
KNN = 20
NSEG = 8
NCOMP = 512
CH = 512
NCHUNK = N // CH
RT = 128
RE = 128
RP = 1024
KPAD = 32
NEG_BIG = -3.0e38


def _elu(v):
    return jnp.where(v > 0, v, jnp.exp(jnp.minimum(v, 0.0)) - 1.0)


def _ffm_body(x_ref, w_ref, b_ref, o_ref):
    o_ref[...] = jnp.dot(x_ref[...], w_ref[...],
                         preferred_element_type=jnp.float32) + b_ref[...]


def _ffm(xp, wp, b):
    return pl.pallas_call(
        _ffm_body,
        grid=(N // RP,),
        in_specs=[pl.BlockSpec((RP, 8), lambda i: (i, 0)),
                  pl.BlockSpec((8, HID), lambda i: (0, 0)),
                  pl.BlockSpec((1, HID), lambda i: (0, 0))],
        out_specs=pl.BlockSpec((RP, HID), lambda i: (i, 0)),
        out_shape=jax.ShapeDtypeStruct((N, HID), jnp.float32),
    )(xp, wp, b)


def _pre_body(h_ref, w1_ref, b1_ref, a_ref, bmat_ref, sq_ref):
    h = h_ref[...]
    w1a = w1_ref[:HID, :]
    w1b = w1_ref[HID:, :]
    a_ref[...] = jnp.dot(h, w1a - w1b,
                         preferred_element_type=jnp.float32) + b1_ref[...]
    bmat_ref[:, :HID] = jnp.dot(h, w1b, preferred_element_type=jnp.float32)
    sq_ref[...] = jnp.sum(h * h, axis=1, keepdims=True)


def _pre(h, w1, b1):
    return pl.pallas_call(
        _pre_body,
        grid=(N // RP,),
        in_specs=[pl.BlockSpec((RP, HID), lambda i: (i, 0)),
                  pl.BlockSpec((2 * HID, HID), lambda i: (0, 0)),
                  pl.BlockSpec((1, HID), lambda i: (0, 0))],
        out_specs=[pl.BlockSpec((RP, HID), lambda i: (i, 0)),
                   pl.BlockSpec((RP, 2 * HID), lambda i: (i, 0)),
                   pl.BlockSpec((RP, 1), lambda i: (i, 0))],
        out_shape=[jax.ShapeDtypeStruct((N, HID), jnp.float32),
                   jax.ShapeDtypeStruct((N, 2 * HID), jnp.float32),
                   jax.ShapeDtypeStruct((N, 1), jnp.float32)],
    )(h, w1, b1)


def _topk_body(bounds_ref, hr_ref, hall_ref, sqr_ref, sqc_ref, br_ref,
               bc_ref, idx_ref, d_ref):
    i = pl.program_id(0)
    c0 = bounds_ref[i, 0]
    c1 = bounds_ref[i, 1]
    hr = hr_ref[0]
    sqr = sqr_ref[0]
    br = br_ref[0]

    def fill(c, _):
        hc = hall_ref[c]
        dot = lax.dot_general(hc, hr, (((1,), (1,)), ((), ())),
                              preferred_element_type=jnp.float32)
        d = (sqr - 2.0 * dot) + sqc_ref[c]
        d = jnp.where(br != bc_ref[c], jnp.inf, d)
        d_ref[c] = d
        return 0

    lax.fori_loop(c0, c1, fill, 0)

    col = lax.broadcasted_iota(jnp.int32, (CH, RT), 0)
    big = jnp.int32(2 ** 30)
    inf1 = jnp.full((1, RT), jnp.inf, jnp.float32)
    big1 = jnp.full((1, RT), big, jnp.int32)

    def lexlt(a, ia, b, ib):
        return (a < b) | ((a == b) & (ia < ib))

    prev1 = jnp.full((1, RT), -1, jnp.int32)
    prev2 = jnp.full((1, RT), -1, jnp.int32)
    picks = []
    for _ in range(KNN // 2):
        def scan(c, carry):
            m1, i1, m2, i2 = carry
            v = d_ref[c]
            colg = col + c * CH
            v = jnp.where((colg == prev1) | (colg == prev2), jnp.inf, v)
            d_ref[c] = v
            mc1 = jnp.min(v, axis=0, keepdims=True)
            ic1 = jnp.min(jnp.where(v == mc1, colg, big), axis=0,
                          keepdims=True)
            vx = jnp.where(colg == ic1, jnp.inf, v)
            mc2 = jnp.min(vx, axis=0, keepdims=True)
            ic2 = jnp.min(jnp.where(vx == mc2, colg, big), axis=0,
                          keepdims=True)
            cw = lexlt(mc1, ic1, m1, i1)
            nm1 = jnp.where(cw, mc1, m1)
            ni1 = jnp.where(cw, ic1, i1)
            lm = jnp.where(cw, m1, mc1)
            li = jnp.where(cw, i1, ic1)
            ws = jnp.where(cw, mc2, m2)
            wi = jnp.where(cw, ic2, i2)
            sw = lexlt(lm, li, ws, wi)
            nm2 = jnp.where(sw, lm, ws)
            ni2 = jnp.where(sw, li, wi)
            return nm1, ni1, nm2, ni2

        _, i1, _, i2 = lax.fori_loop(c0, c1, scan,
                                     (inf1, big1, inf1, big1))
        picks.append(jnp.clip(i1, 0, N - 1))
        picks.append(jnp.clip(i2, 0, N - 1))
        prev1 = i1
        prev2 = i2
    pad = [picks[-1]] * (KPAD - len(picks))
    idx_ref[0] = jnp.concatenate(picks + pad, axis=0)


def _topk(h3, sqrT, sqc, brT, bc, bounds):
    nb = N // RT
    grid_spec = pltpu.PrefetchScalarGridSpec(
        num_scalar_prefetch=1,
        grid=(nb,),
        in_specs=[pl.BlockSpec((1, RT, HID), lambda i, s: (i, 0, 0)),
                  pl.BlockSpec((NCHUNK, CH, HID), lambda i, s: (0, 0, 0)),
                  pl.BlockSpec((1, 1, RT), lambda i, s: (i, 0, 0)),
                  pl.BlockSpec((NCHUNK, CH, 1), lambda i, s: (0, 0, 0)),
                  pl.BlockSpec((1, 1, RT), lambda i, s: (i, 0, 0)),
                  pl.BlockSpec((NCHUNK, CH, 1), lambda i, s: (0, 0, 0))],
        out_specs=pl.BlockSpec((1, KPAD, RT), lambda i, s: (i, 0, 0)),
        scratch_shapes=[pltpu.VMEM((NCHUNK, CH, RT), jnp.float32)],
    )
    hrows = h3.reshape(nb, RT, HID)
    return pl.pallas_call(
        _topk_body,
        grid_spec=grid_spec,
        out_shape=jax.ShapeDtypeStruct((nb, KPAD, RT), jnp.int32),
    )(bounds, hrows, h3, sqrT, sqc, brT, bc)


def _topk_fixed(h, sq, bf32, bounds):
    nb = N // RT
    return _topk(h.reshape(NCHUNK, CH, HID),
                 sq.reshape(nb, 1, RT),
                 sq.reshape(NCHUNK, CH, 1),
                 bf32.reshape(nb, 1, RT),
                 bf32.reshape(NCHUNK, CH, 1),
                 bounds)


def _make_gather():
    info = plsc.get_sparse_core_info()
    nw = info.num_cores * info.num_subcores
    total = N * KNN
    bpw = total // nw
    chunk = 512
    nchunks = bpw // chunk
    mesh = plsc.VectorSubcoreMesh(core_axis_name="c", subcore_axis_name="s")

    @functools.partial(
        pl.kernel, mesh=mesh,
        out_type=jax.ShapeDtypeStruct((total, 2 * HID), jnp.float32),
        scratch_types=[pltpu.VMEM((bpw,), jnp.int32),
                       pltpu.VMEM((chunk, 2 * HID), jnp.float32),
                       pltpu.SemaphoreType.DMA],
    )
    def gather(table_hbm, idx_hbm, out_hbm, idx_v, rows_v, sem):
        wid = lax.axis_index("s") * info.num_cores + lax.axis_index("c")
        base = wid * bpw
        pltpu.sync_copy(idx_hbm.at[pl.ds(base, bpw)], idx_v)
        for t in range(nchunks):
            idx_slice = idx_v.at[pl.ds(t * chunk, chunk)]
            pltpu.async_copy(table_hbm.at[idx_slice], rows_v, sem).wait()
            pltpu.sync_copy(rows_v,
                            out_hbm.at[pl.ds(base + t * chunk, chunk)])

    return gather


def _edge_body(a_ref, bg_ref, h_ref, w2_ref, b2_ref, w3_ref, b3_ref,
               g_ref, bl_ref, wl_ref, bll_ref, rc_ref, o_ref):
    a = a_ref[...]
    ae = jnp.broadcast_to(a[None, :, :], (KNN, RE, HID)).reshape(
        KNN * RE, HID)
    m1 = _elu(ae + bg_ref[:, :HID])
    m2 = _elu(jnp.dot(m1, w2_ref[...],
                      preferred_element_type=jnp.float32) + b2_ref[...])
    m3 = jnp.dot(m2, w3_ref[...],
                 preferred_element_type=jnp.float32) + b3_ref[...]
    agg = jnp.max(m3.reshape(KNN, RE, HID), axis=0)
    mu = jnp.mean(agg, axis=1, keepdims=True)
    var = jnp.mean((agg - mu) ** 2, axis=1, keepdims=True)
    y = (agg - mu) / jnp.sqrt(var + 1e-5) * g_ref[...] + bl_ref[...]
    o = jnp.dot(_elu(y), wl_ref[...],
                preferred_element_type=jnp.float32) + bll_ref[...]
    o_ref[...] = h_ref[...] + rc_ref[0, 0] * o


def _edge(a, bg, h, w2, b2, w3, b3, g, bln, wl, bl, rc):
    return pl.pallas_call(
        _edge_body,
        grid=(N // RE,),
        in_specs=[pl.BlockSpec((RE, HID), lambda i: (i, 0)),
                  pl.BlockSpec((RE * KNN, 2 * HID), lambda i: (i, 0)),
                  pl.BlockSpec((RE, HID), lambda i: (i, 0)),
                  pl.BlockSpec((HID, HID), lambda i: (0, 0)),
                  pl.BlockSpec((1, HID), lambda i: (0, 0)),
                  pl.BlockSpec((HID, HID), lambda i: (0, 0)),
                  pl.BlockSpec((1, HID), lambda i: (0, 0)),
                  pl.BlockSpec((1, HID), lambda i: (0, 0)),
                  pl.BlockSpec((1, HID), lambda i: (0, 0)),
                  pl.BlockSpec((HID, HID), lambda i: (0, 0)),
                  pl.BlockSpec((1, HID), lambda i: (0, 0)),
                  pl.BlockSpec((1, 1), lambda i: (0, 0))],
        out_specs=pl.BlockSpec((RE, HID), lambda i: (i, 0)),
        out_shape=jax.ShapeDtypeStruct((N, HID), jnp.float32),
    )(a, bg, h, w2, b2, w3, b3, g, bln, wl, bl, rc)


def _reg_body(h_ref, br_ref, w0_ref, b0_ref, w1_ref, b1_ref, w2_ref,
              b2_ref, w3_ref, b3_ref, o_ref, pool_ref):
    i = pl.program_id(0)
    nb = pl.num_programs(0)

    @pl.when(i == 0)
    def _():
        pool_ref[...] = jnp.full((NSEG, HID), NEG_BIG, jnp.float32)

    r = jnp.dot(h_ref[...], w0_ref[...],
                preferred_element_type=jnp.float32) + b0_ref[...]
    br = br_ref[...]
    for s in range(NSEG):
        rs = jnp.where(br == jnp.float32(s), r, NEG_BIG)
        ps = jnp.max(rs, axis=0, keepdims=True)
        pool_ref[pl.ds(s, 1), :] = jnp.maximum(pool_ref[pl.ds(s, 1), :], ps)

    @pl.when(i == nb - 1)
    def _():
        p = pool_ref[...]
        t = _elu(jnp.dot(p, w1_ref[...],
                         preferred_element_type=jnp.float32) + b1_ref[...])
        t = _elu(jnp.dot(t, w2_ref[...],
                         preferred_element_type=jnp.float32) + b2_ref[...])
        o_ref[...] = jnp.dot(t, w3_ref[...],
                             preferred_element_type=jnp.float32) + b3_ref[...]


def _reg(h, br, ws):
    (w0, b0), (w1, b1), (w2, b2), (w3, b3) = ws
    return pl.pallas_call(
        _reg_body,
        grid=(N // RP,),
        in_specs=[pl.BlockSpec((RP, HID), lambda i: (i, 0)),
                  pl.BlockSpec((RP, 1), lambda i: (i, 0)),
                  pl.BlockSpec((HID, HID), lambda i: (0, 0)),
                  pl.BlockSpec((1, HID), lambda i: (0, 0)),
                  pl.BlockSpec((HID, HID), lambda i: (0, 0)),
                  pl.BlockSpec((1, HID), lambda i: (0, 0)),
                  pl.BlockSpec((HID, HID), lambda i: (0, 0)),
                  pl.BlockSpec((1, HID), lambda i: (0, 0)),
                  pl.BlockSpec((HID, NCOMP), lambda i: (0, 0)),
                  pl.BlockSpec((1, NCOMP), lambda i: (0, 0))],
        out_specs=pl.BlockSpec((NSEG, NCOMP), lambda i: (0, 0)),
        out_shape=jax.ShapeDtypeStruct((NSEG, NCOMP), jnp.float32),
        scratch_shapes=[pltpu.VMEM((NSEG, HID), jnp.float32)],
    )(h, br, w0, b0, w1, b1, w2, b2, w3, b3)


def kernel(x, batch, params):
    xp = jnp.pad(x, ((0, 0), (0, 5)))
    wf = jnp.pad(params['ffm'][0], ((0, 5), (0, 0)))
    bf = params['ffm'][1].reshape(1, HID)
    h = _ffm(xp, wf, bf)

    batch = batch.astype(jnp.int32)
    bf32 = batch.astype(jnp.float32)
    br_rows = bf32.reshape(N, 1)

    seg_ids = jnp.arange(NSEG, dtype=batch.dtype)
    seg_start = jnp.searchsorted(batch, seg_ids, side='left')
    seg_end = jnp.searchsorted(batch, seg_ids, side='right')
    b_lo = batch[::RT]
    b_hi = batch[RT - 1::RT]
    c0 = (seg_start[b_lo] // CH).astype(jnp.int32)
    c1 = ((seg_end[b_hi] + CH - 1) // CH).astype(jnp.int32)
    bounds = jnp.stack([c0, c1], axis=1)

    gather = _make_gather()

    for lp in params['layers']:
        a, bmat, sq = _pre(h, lp['mlp1'][0], lp['mlp1'][1].reshape(1, HID))
        idx = _topk_fixed(h, sq, bf32, bounds)
        idxf = idx[:, :KNN, :].reshape(N * KNN)
        bg = gather(bmat, idxf)
        h = _edge(a, bg, h,
                  lp['mlp2'][0], lp['mlp2'][1].reshape(1, HID),
                  lp['mlp3'][0], lp['mlp3'][1].reshape(1, HID),
                  lp['ln_g'].reshape(1, HID), lp['ln_b'].reshape(1, HID),
                  lp['lin'][0], lp['lin'][1].reshape(1, HID),
                  lp['rc'].reshape(1, 1))

    ws = [(params['reg'][j][0],
           params['reg'][j][1].reshape(1, -1)) for j in range(4)]
    return _reg(h, br_rows, ws)


# speedup vs baseline: 1.1245x; 1.0459x over previous
import functools

import jax
import jax.numpy as jnp
from jax import lax
from jax.experimental import pallas as pl
from jax.experimental.pallas import tpu as pltpu
from jax.experimental.pallas import tpu_sc as plsc

N = 8192
HID = 64
KNN = 20
NSEG = 8
NCOMP = 512
CH = 256
NCHUNK = N // CH
RT = 128
RE = 128
RP = 1024
KPAD = 32
NEG_BIG = -3.0e38


def _elu(v):
    return jnp.where(v > 0, v, jnp.exp(jnp.minimum(v, 0.0)) - 1.0)


def _ffm_body(x_ref, w_ref, b_ref, o_ref):
    o_ref[...] = jnp.dot(x_ref[...], w_ref[...],
                         preferred_element_type=jnp.float32) + b_ref[...]


def _ffm(xp, wp, b):
    return pl.pallas_call(
        _ffm_body,
        grid=(N // RP,),
        in_specs=[pl.BlockSpec((RP, 8), lambda i: (i, 0)),
                  pl.BlockSpec((8, HID), lambda i: (0, 0)),
                  pl.BlockSpec((1, HID), lambda i: (0, 0))],
        out_specs=pl.BlockSpec((RP, HID), lambda i: (i, 0)),
        out_shape=jax.ShapeDtypeStruct((N, HID), jnp.float32),
    )(xp, wp, b)


def _pre_body(h_ref, w1_ref, b1_ref, a_ref, bmat_ref, sq_ref):
    h = h_ref[...]
    w1a = w1_ref[:HID, :]
    w1b = w1_ref[HID:, :]
    a_ref[...] = jnp.dot(h, w1a - w1b,
                         preferred_element_type=jnp.float32) + b1_ref[...]
    bmat_ref[:, :HID] = jnp.dot(h, w1b, preferred_element_type=jnp.float32)
    sq_ref[...] = jnp.sum(h * h, axis=1, keepdims=True)


def _pre(h, w1, b1):
    return pl.pallas_call(
        _pre_body,
        grid=(N // RP,),
        in_specs=[pl.BlockSpec((RP, HID), lambda i: (i, 0)),
                  pl.BlockSpec((2 * HID, HID), lambda i: (0, 0)),
                  pl.BlockSpec((1, HID), lambda i: (0, 0))],
        out_specs=[pl.BlockSpec((RP, HID), lambda i: (i, 0)),
                   pl.BlockSpec((RP, 2 * HID), lambda i: (i, 0)),
                   pl.BlockSpec((RP, 1), lambda i: (i, 0))],
        out_shape=[jax.ShapeDtypeStruct((N, HID), jnp.float32),
                   jax.ShapeDtypeStruct((N, 2 * HID), jnp.float32),
                   jax.ShapeDtypeStruct((N, 1), jnp.float32)],
    )(h, w1, b1)


def _topk_body(bounds_ref, hr_ref, hall_ref, sqr_ref, sqc_ref, br_ref,
               bc_ref, idx_ref, d_ref):
    i = pl.program_id(0)
    c0 = bounds_ref[i, 0]
    c1 = bounds_ref[i, 1]
    hr = hr_ref[0]
    sqr = sqr_ref[0]
    br = br_ref[0]

    def fill(c, _):
        hc = hall_ref[c]
        dot = lax.dot_general(hc, hr, (((1,), (1,)), ((), ())),
                              preferred_element_type=jnp.float32)
        d = (sqr - 2.0 * dot) + sqc_ref[c]
        d = jnp.where(br != bc_ref[c], jnp.inf, d)
        d_ref[c] = d
        return 0

    lax.fori_loop(c0, c1, fill, 0)

    col = lax.broadcasted_iota(jnp.int32, (CH, RT), 0)
    big = jnp.int32(2 ** 30)
    inf1 = jnp.full((1, RT), jnp.inf, jnp.float32)
    big1 = jnp.full((1, RT), big, jnp.int32)

    def lexlt(a, ia, b, ib):
        return (a < b) | ((a == b) & (ia < ib))

    prev1 = jnp.full((1, RT), -1, jnp.int32)
    prev2 = jnp.full((1, RT), -1, jnp.int32)
    picks = []
    for _ in range(KNN // 2):
        def scan(c, carry):
            m1, i1, m2, i2 = carry
            v = d_ref[c]
            colg = col + c * CH
            v = jnp.where((colg == prev1) | (colg == prev2), jnp.inf, v)
            d_ref[c] = v
            mc1 = jnp.min(v, axis=0, keepdims=True)
            il1 = jnp.argmin(v, axis=0).astype(jnp.int32).reshape(1, RT)
            ic1 = il1 + c * CH
            vx = jnp.where(col == il1, jnp.inf, v)
            mc2 = jnp.min(vx, axis=0, keepdims=True)
            ic2 = jnp.argmin(vx, axis=0).astype(jnp.int32).reshape(
                1, RT) + c * CH
            cw = lexlt(mc1, ic1, m1, i1)
            nm1 = jnp.where(cw, mc1, m1)
            ni1 = jnp.where(cw, ic1, i1)
            lm = jnp.where(cw, m1, mc1)
            li = jnp.where(cw, i1, ic1)
            ws = jnp.where(cw, mc2, m2)
            wi = jnp.where(cw, ic2, i2)
            sw = lexlt(lm, li, ws, wi)
            nm2 = jnp.where(sw, lm, ws)
            ni2 = jnp.where(sw, li, wi)
            return nm1, ni1, nm2, ni2

        _, i1, _, i2 = lax.fori_loop(c0, c1, scan,
                                     (inf1, big1, inf1, big1))
        picks.append(jnp.clip(i1, 0, N - 1))
        picks.append(jnp.clip(i2, 0, N - 1))
        prev1 = i1
        prev2 = i2
    pad = [picks[-1]] * (KPAD - len(picks))
    idx_ref[0] = jnp.concatenate(picks + pad, axis=0)


def _topk(h3, sqrT, sqc, brT, bc, bounds):
    nb = N // RT
    grid_spec = pltpu.PrefetchScalarGridSpec(
        num_scalar_prefetch=1,
        grid=(nb,),
        in_specs=[pl.BlockSpec((1, RT, HID), lambda i, s: (i, 0, 0)),
                  pl.BlockSpec((NCHUNK, CH, HID), lambda i, s: (0, 0, 0)),
                  pl.BlockSpec((1, 1, RT), lambda i, s: (i, 0, 0)),
                  pl.BlockSpec((NCHUNK, CH, 1), lambda i, s: (0, 0, 0)),
                  pl.BlockSpec((1, 1, RT), lambda i, s: (i, 0, 0)),
                  pl.BlockSpec((NCHUNK, CH, 1), lambda i, s: (0, 0, 0))],
        out_specs=pl.BlockSpec((1, KPAD, RT), lambda i, s: (i, 0, 0)),
        scratch_shapes=[pltpu.VMEM((NCHUNK, CH, RT), jnp.float32)],
    )
    hrows = h3.reshape(nb, RT, HID)
    return pl.pallas_call(
        _topk_body,
        grid_spec=grid_spec,
        out_shape=jax.ShapeDtypeStruct((nb, KPAD, RT), jnp.int32),
    )(bounds, hrows, h3, sqrT, sqc, brT, bc)


def _topk_fixed(h, sq, bf32, bounds):
    nb = N // RT
    return _topk(h.reshape(NCHUNK, CH, HID),
                 sq.reshape(nb, 1, RT),
                 sq.reshape(NCHUNK, CH, 1),
                 bf32.reshape(nb, 1, RT),
                 bf32.reshape(NCHUNK, CH, 1),
                 bounds)


def _make_gather():
    info = plsc.get_sparse_core_info()
    nw = info.num_cores * info.num_subcores
    total = N * KNN
    bpw = total // nw
    chunk = 512
    nchunks = bpw // chunk
    mesh = plsc.VectorSubcoreMesh(core_axis_name="c", subcore_axis_name="s")

    @functools.partial(
        pl.kernel, mesh=mesh,
        out_type=jax.ShapeDtypeStruct((total, 2 * HID), jnp.float32),
        scratch_types=[pltpu.VMEM((bpw,), jnp.int32),
                       pltpu.VMEM((chunk, 2 * HID), jnp.float32),
                       pltpu.SemaphoreType.DMA],
    )
    def gather(table_hbm, idx_hbm, out_hbm, idx_v, rows_v, sem):
        wid = lax.axis_index("s") * info.num_cores + lax.axis_index("c")
        base = wid * bpw
        pltpu.sync_copy(idx_hbm.at[pl.ds(base, bpw)], idx_v)
        for t in range(nchunks):
            idx_slice = idx_v.at[pl.ds(t * chunk, chunk)]
            pltpu.async_copy(table_hbm.at[idx_slice], rows_v, sem).wait()
            pltpu.sync_copy(rows_v,
                            out_hbm.at[pl.ds(base + t * chunk, chunk)])

    return gather


def _edge_body(a_ref, bg_ref, h_ref, w2_ref, b2_ref, w3_ref, b3_ref,
               g_ref, bl_ref, wl_ref, bll_ref, rc_ref, o_ref):
    a = a_ref[...]
    ae = jnp.broadcast_to(a[None, :, :], (KNN, RE, HID)).reshape(
        KNN * RE, HID)
    m1 = _elu(ae + bg_ref[:, :HID])
    m2 = _elu(jnp.dot(m1, w2_ref[...],
                      preferred_element_type=jnp.float32) + b2_ref[...])
    m3 = jnp.dot(m2, w3_ref[...],
                 preferred_element_type=jnp.float32) + b3_ref[...]
    agg = jnp.max(m3.reshape(KNN, RE, HID), axis=0)
    mu = jnp.mean(agg, axis=1, keepdims=True)
    var = jnp.mean((agg - mu) ** 2, axis=1, keepdims=True)
    y = (agg - mu) / jnp.sqrt(var + 1e-5) * g_ref[...] + bl_ref[...]
    o = jnp.dot(_elu(y), wl_ref[...],
                preferred_element_type=jnp.float32) + bll_ref[...]
    o_ref[...] = h_ref[...] + rc_ref[0, 0] * o


def _edge(a, bg, h, w2, b2, w3, b3, g, bln, wl, bl, rc):
    return pl.pallas_call(
        _edge_body,
        grid=(N // RE,),
        in_specs=[pl.BlockSpec((RE, HID), lambda i: (i, 0)),
                  pl.BlockSpec((RE * KNN, 2 * HID), lambda i: (i, 0)),
                  pl.BlockSpec((RE, HID), lambda i: (i, 0)),
                  pl.BlockSpec((HID, HID), lambda i: (0, 0)),
                  pl.BlockSpec((1, HID), lambda i: (0, 0)),
                  pl.BlockSpec((HID, HID), lambda i: (0, 0)),
                  pl.BlockSpec((1, HID), lambda i: (0, 0)),
                  pl.BlockSpec((1, HID), lambda i: (0, 0)),
                  pl.BlockSpec((1, HID), lambda i: (0, 0)),
                  pl.BlockSpec((HID, HID), lambda i: (0, 0)),
                  pl.BlockSpec((1, HID), lambda i: (0, 0)),
                  pl.BlockSpec((1, 1), lambda i: (0, 0))],
        out_specs=pl.BlockSpec((RE, HID), lambda i: (i, 0)),
        out_shape=jax.ShapeDtypeStruct((N, HID), jnp.float32),
    )(a, bg, h, w2, b2, w3, b3, g, bln, wl, bl, rc)


def _reg_body(h_ref, br_ref, w0_ref, b0_ref, w1_ref, b1_ref, w2_ref,
              b2_ref, w3_ref, b3_ref, o_ref, pool_ref):
    i = pl.program_id(0)
    nb = pl.num_programs(0)

    @pl.when(i == 0)
    def _():
        pool_ref[...] = jnp.full((NSEG, HID), NEG_BIG, jnp.float32)

    r = jnp.dot(h_ref[...], w0_ref[...],
                preferred_element_type=jnp.float32) + b0_ref[...]
    br = br_ref[...]
    for s in range(NSEG):
        rs = jnp.where(br == jnp.float32(s), r, NEG_BIG)
        ps = jnp.max(rs, axis=0, keepdims=True)
        pool_ref[pl.ds(s, 1), :] = jnp.maximum(pool_ref[pl.ds(s, 1), :], ps)

    @pl.when(i == nb - 1)
    def _():
        p = pool_ref[...]
        t = _elu(jnp.dot(p, w1_ref[...],
                         preferred_element_type=jnp.float32) + b1_ref[...])
        t = _elu(jnp.dot(t, w2_ref[...],
                         preferred_element_type=jnp.float32) + b2_ref[...])
        o_ref[...] = jnp.dot(t, w3_ref[...],
                             preferred_element_type=jnp.float32) + b3_ref[...]


def _reg(h, br, ws):
    (w0, b0), (w1, b1), (w2, b2), (w3, b3) = ws
    return pl.pallas_call(
        _reg_body,
        grid=(N // RP,),
        in_specs=[pl.BlockSpec((RP, HID), lambda i: (i, 0)),
                  pl.BlockSpec((RP, 1), lambda i: (i, 0)),
                  pl.BlockSpec((HID, HID), lambda i: (0, 0)),
                  pl.BlockSpec((1, HID), lambda i: (0, 0)),
                  pl.BlockSpec((HID, HID), lambda i: (0, 0)),
                  pl.BlockSpec((1, HID), lambda i: (0, 0)),
                  pl.BlockSpec((HID, HID), lambda i: (0, 0)),
                  pl.BlockSpec((1, HID), lambda i: (0, 0)),
                  pl.BlockSpec((HID, NCOMP), lambda i: (0, 0)),
                  pl.BlockSpec((1, NCOMP), lambda i: (0, 0))],
        out_specs=pl.BlockSpec((NSEG, NCOMP), lambda i: (0, 0)),
        out_shape=jax.ShapeDtypeStruct((NSEG, NCOMP), jnp.float32),
        scratch_shapes=[pltpu.VMEM((NSEG, HID), jnp.float32)],
    )(h, br, w0, b0, w1, b1, w2, b2, w3, b3)


def kernel(x, batch, params):
    xp = jnp.pad(x, ((0, 0), (0, 5)))
    wf = jnp.pad(params['ffm'][0], ((0, 5), (0, 0)))
    bf = params['ffm'][1].reshape(1, HID)
    h = _ffm(xp, wf, bf)

    batch = batch.astype(jnp.int32)
    bf32 = batch.astype(jnp.float32)
    br_rows = bf32.reshape(N, 1)

    seg_ids = jnp.arange(NSEG, dtype=batch.dtype)
    seg_start = jnp.searchsorted(batch, seg_ids, side='left')
    seg_end = jnp.searchsorted(batch, seg_ids, side='right')
    b_lo = batch[::RT]
    b_hi = batch[RT - 1::RT]
    c0 = (seg_start[b_lo] // CH).astype(jnp.int32)
    c1 = ((seg_end[b_hi] + CH - 1) // CH).astype(jnp.int32)
    bounds = jnp.stack([c0, c1], axis=1)

    gather = _make_gather()

    for lp in params['layers']:
        a, bmat, sq = _pre(h, lp['mlp1'][0], lp['mlp1'][1].reshape(1, HID))
        idx = _topk_fixed(h, sq, bf32, bounds)
        idxf = idx[:, :KNN, :].reshape(N * KNN)
        bg = gather(bmat, idxf)
        h = _edge(a, bg, h,
                  lp['mlp2'][0], lp['mlp2'][1].reshape(1, HID),
                  lp['mlp3'][0], lp['mlp3'][1].reshape(1, HID),
                  lp['ln_g'].reshape(1, HID), lp['ln_b'].reshape(1, HID),
                  lp['lin'][0], lp['lin'][1].reshape(1, HID),
                  lp['rc'].reshape(1, 1))

    ws = [(params['reg'][j][0],
           params['reg'][j][1].reshape(1, -1)) for j in range(4)]
    return _reg(h, br_rows, ws)


# speedup vs baseline: 1.1378x; 1.0118x over previous
import functools

import jax
import jax.numpy as jnp
from jax import lax
from jax.experimental import pallas as pl
from jax.experimental.pallas import tpu as pltpu
from jax.experimental.pallas import tpu_sc as plsc

N = 8192
HID = 64
KNN = 20
NSEG = 8
NCOMP = 512
CH = 256
NCHUNK = N // CH
RT = 128
RE = 128
RP = 1024
KPAD = 32
NEG_BIG = -3.0e38


def _elu(v):
    return jnp.where(v > 0, v, jnp.exp(jnp.minimum(v, 0.0)) - 1.0)


def _ffm_body(x_ref, w_ref, b_ref, o_ref):
    o_ref[...] = jnp.dot(x_ref[...], w_ref[...],
                         preferred_element_type=jnp.float32) + b_ref[...]


def _ffm(xp, wp, b):
    return pl.pallas_call(
        _ffm_body,
        grid=(N // RP,),
        in_specs=[pl.BlockSpec((RP, 8), lambda i: (i, 0)),
                  pl.BlockSpec((8, HID), lambda i: (0, 0)),
                  pl.BlockSpec((1, HID), lambda i: (0, 0))],
        out_specs=pl.BlockSpec((RP, HID), lambda i: (i, 0)),
        out_shape=jax.ShapeDtypeStruct((N, HID), jnp.float32),
    )(xp, wp, b)


def _pre_body(h_ref, w1_ref, b1_ref, a_ref, bmat_ref, sq_ref):
    h = h_ref[...]
    w1a = w1_ref[:HID, :]
    w1b = w1_ref[HID:, :]
    a_ref[...] = jnp.dot(h, w1a - w1b,
                         preferred_element_type=jnp.float32) + b1_ref[...]
    bmat_ref[:, :HID] = jnp.dot(h, w1b, preferred_element_type=jnp.float32)
    sq_ref[...] = jnp.sum(h * h, axis=1, keepdims=True)


def _pre(h, w1, b1):
    return pl.pallas_call(
        _pre_body,
        grid=(N // RP,),
        in_specs=[pl.BlockSpec((RP, HID), lambda i: (i, 0)),
                  pl.BlockSpec((2 * HID, HID), lambda i: (0, 0)),
                  pl.BlockSpec((1, HID), lambda i: (0, 0))],
        out_specs=[pl.BlockSpec((RP, HID), lambda i: (i, 0)),
                   pl.BlockSpec((RP, 2 * HID), lambda i: (i, 0)),
                   pl.BlockSpec((RP, 1), lambda i: (i, 0))],
        out_shape=[jax.ShapeDtypeStruct((N, HID), jnp.float32),
                   jax.ShapeDtypeStruct((N, 2 * HID), jnp.float32),
                   jax.ShapeDtypeStruct((N, 1), jnp.float32)],
    )(h, w1, b1)


def _topk_body(bounds_ref, hr_ref, hall_ref, sqr_ref, sqc_ref, br_ref,
               bc_ref, idx_ref, d_ref):
    i = pl.program_id(0)
    c0 = bounds_ref[i, 0]
    c1 = bounds_ref[i, 1]
    hr = hr_ref[0]
    sqr = sqr_ref[0]
    br = br_ref[0]

    col = lax.broadcasted_iota(jnp.int32, (CH, RT), 0)
    big = jnp.int32(2 ** 30)
    inf1 = jnp.full((1, RT), jnp.inf, jnp.float32)
    big1 = jnp.full((1, RT), big, jnp.int32)

    def lexlt(a, ia, b, ib):
        return (a < b) | ((a == b) & (ia < ib))

    def extract2(v, c, carry):
        m1, i1, m2, i2 = carry
        mc1 = jnp.min(v, axis=0, keepdims=True)
        il1 = jnp.argmin(v, axis=0).astype(jnp.int32).reshape(1, RT)
        ic1 = il1 + c * CH
        vx = jnp.where(col == il1, jnp.inf, v)
        mc2 = jnp.min(vx, axis=0, keepdims=True)
        ic2 = jnp.argmin(vx, axis=0).astype(jnp.int32).reshape(
            1, RT) + c * CH
        cw = lexlt(mc1, ic1, m1, i1)
        nm1 = jnp.where(cw, mc1, m1)
        ni1 = jnp.where(cw, ic1, i1)
        lm = jnp.where(cw, m1, mc1)
        li = jnp.where(cw, i1, ic1)
        ws = jnp.where(cw, mc2, m2)
        wi = jnp.where(cw, ic2, i2)
        sw = lexlt(lm, li, ws, wi)
        nm2 = jnp.where(sw, lm, ws)
        ni2 = jnp.where(sw, li, wi)
        return nm1, ni1, nm2, ni2

    def fill(c, carry):
        hc = hall_ref[c]
        dot = lax.dot_general(hc, hr, (((1,), (1,)), ((), ())),
                              preferred_element_type=jnp.float32)
        d = (sqr - 2.0 * dot) + sqc_ref[c]
        d = jnp.where(br != bc_ref[c], jnp.inf, d)
        d_ref[c] = d
        return extract2(d, c, carry)

    _, i1, _, i2 = lax.fori_loop(c0, c1, fill, (inf1, big1, inf1, big1))
    picks = [jnp.clip(i1, 0, N - 1), jnp.clip(i2, 0, N - 1)]
    prev1, prev2 = i1, i2
    npass = KNN // 2
    for p in range(1, npass):
        def scan(c, carry):
            v = d_ref[c]
            colg = col + c * CH
            v = jnp.where((colg == prev1) | (colg == prev2), jnp.inf, v)
            if p < npass - 1:
                d_ref[c] = v
            return extract2(v, c, carry)

        _, i1, _, i2 = lax.fori_loop(c0, c1, scan,
                                     (inf1, big1, inf1, big1))
        picks.append(jnp.clip(i1, 0, N - 1))
        picks.append(jnp.clip(i2, 0, N - 1))
        prev1 = i1
        prev2 = i2
    pad = [picks[-1]] * (KPAD - len(picks))
    idx_ref[0] = jnp.concatenate(picks + pad, axis=0)


def _topk(h3, sqrT, sqc, brT, bc, bounds):
    nb = N // RT
    grid_spec = pltpu.PrefetchScalarGridSpec(
        num_scalar_prefetch=1,
        grid=(nb,),
        in_specs=[pl.BlockSpec((1, RT, HID), lambda i, s: (i, 0, 0)),
                  pl.BlockSpec((NCHUNK, CH, HID), lambda i, s: (0, 0, 0)),
                  pl.BlockSpec((1, 1, RT), lambda i, s: (i, 0, 0)),
                  pl.BlockSpec((NCHUNK, CH, 1), lambda i, s: (0, 0, 0)),
                  pl.BlockSpec((1, 1, RT), lambda i, s: (i, 0, 0)),
                  pl.BlockSpec((NCHUNK, CH, 1), lambda i, s: (0, 0, 0))],
        out_specs=pl.BlockSpec((1, KPAD, RT), lambda i, s: (i, 0, 0)),
        scratch_shapes=[pltpu.VMEM((NCHUNK, CH, RT), jnp.float32)],
    )
    hrows = h3.reshape(nb, RT, HID)
    return pl.pallas_call(
        _topk_body,
        grid_spec=grid_spec,
        out_shape=jax.ShapeDtypeStruct((nb, KPAD, RT), jnp.int32),
    )(bounds, hrows, h3, sqrT, sqc, brT, bc)


def _topk_fixed(h, sq, bf32, bounds):
    nb = N // RT
    return _topk(h.reshape(NCHUNK, CH, HID),
                 sq.reshape(nb, 1, RT),
                 sq.reshape(NCHUNK, CH, 1),
                 bf32.reshape(nb, 1, RT),
                 bf32.reshape(NCHUNK, CH, 1),
                 bounds)


def _make_gather():
    info = plsc.get_sparse_core_info()
    nw = info.num_cores * info.num_subcores
    total = N * KNN
    bpw = total // nw
    chunk = 512
    nchunks = bpw // chunk
    mesh = plsc.VectorSubcoreMesh(core_axis_name="c", subcore_axis_name="s")

    @functools.partial(
        pl.kernel, mesh=mesh,
        out_type=jax.ShapeDtypeStruct((total, 2 * HID), jnp.float32),
        scratch_types=[pltpu.VMEM((bpw,), jnp.int32),
                       pltpu.VMEM((chunk, 2 * HID), jnp.float32),
                       pltpu.SemaphoreType.DMA],
    )
    def gather(table_hbm, idx_hbm, out_hbm, idx_v, rows_v, sem):
        wid = lax.axis_index("s") * info.num_cores + lax.axis_index("c")
        base = wid * bpw
        pltpu.sync_copy(idx_hbm.at[pl.ds(base, bpw)], idx_v)
        for t in range(nchunks):
            idx_slice = idx_v.at[pl.ds(t * chunk, chunk)]
            pltpu.async_copy(table_hbm.at[idx_slice], rows_v, sem).wait()
            pltpu.sync_copy(rows_v,
                            out_hbm.at[pl.ds(base + t * chunk, chunk)])

    return gather


def _edge_body(a_ref, bg_ref, h_ref, w2_ref, b2_ref, w3_ref, b3_ref,
               g_ref, bl_ref, wl_ref, bll_ref, rc_ref, o_ref):
    a = a_ref[...]
    ae = jnp.broadcast_to(a[None, :, :], (KNN, RE, HID)).reshape(
        KNN * RE, HID)
    m1 = _elu(ae + bg_ref[:, :HID])
    m2 = _elu(jnp.dot(m1, w2_ref[...],
                      preferred_element_type=jnp.float32) + b2_ref[...])
    m3 = jnp.dot(m2, w3_ref[...],
                 preferred_element_type=jnp.float32) + b3_ref[...]
    agg = jnp.max(m3.reshape(KNN, RE, HID), axis=0)
    mu = jnp.mean(agg, axis=1, keepdims=True)
    var = jnp.mean((agg - mu) ** 2, axis=1, keepdims=True)
    y = (agg - mu) / jnp.sqrt(var + 1e-5) * g_ref[...] + bl_ref[...]
    o = jnp.dot(_elu(y), wl_ref[...],
                preferred_element_type=jnp.float32) + bll_ref[...]
    o_ref[...] = h_ref[...] + rc_ref[0, 0] * o


def _edge(a, bg, h, w2, b2, w3, b3, g, bln, wl, bl, rc):
    return pl.pallas_call(
        _edge_body,
        grid=(N // RE,),
        in_specs=[pl.BlockSpec((RE, HID), lambda i: (i, 0)),
                  pl.BlockSpec((RE * KNN, 2 * HID), lambda i: (i, 0)),
                  pl.BlockSpec((RE, HID), lambda i: (i, 0)),
                  pl.BlockSpec((HID, HID), lambda i: (0, 0)),
                  pl.BlockSpec((1, HID), lambda i: (0, 0)),
                  pl.BlockSpec((HID, HID), lambda i: (0, 0)),
                  pl.BlockSpec((1, HID), lambda i: (0, 0)),
                  pl.BlockSpec((1, HID), lambda i: (0, 0)),
                  pl.BlockSpec((1, HID), lambda i: (0, 0)),
                  pl.BlockSpec((HID, HID), lambda i: (0, 0)),
                  pl.BlockSpec((1, HID), lambda i: (0, 0)),
                  pl.BlockSpec((1, 1), lambda i: (0, 0))],
        out_specs=pl.BlockSpec((RE, HID), lambda i: (i, 0)),
        out_shape=jax.ShapeDtypeStruct((N, HID), jnp.float32),
    )(a, bg, h, w2, b2, w3, b3, g, bln, wl, bl, rc)


def _reg_body(h_ref, br_ref, w0_ref, b0_ref, w1_ref, b1_ref, w2_ref,
              b2_ref, w3_ref, b3_ref, o_ref, pool_ref):
    i = pl.program_id(0)
    nb = pl.num_programs(0)

    @pl.when(i == 0)
    def _():
        pool_ref[...] = jnp.full((NSEG, HID), NEG_BIG, jnp.float32)

    r = jnp.dot(h_ref[...], w0_ref[...],
                preferred_element_type=jnp.float32) + b0_ref[...]
    br = br_ref[...]
    for s in range(NSEG):
        rs = jnp.where(br == jnp.float32(s), r, NEG_BIG)
        ps = jnp.max(rs, axis=0, keepdims=True)
        pool_ref[pl.ds(s, 1), :] = jnp.maximum(pool_ref[pl.ds(s, 1), :], ps)

    @pl.when(i == nb - 1)
    def _():
        p = pool_ref[...]
        t = _elu(jnp.dot(p, w1_ref[...],
                         preferred_element_type=jnp.float32) + b1_ref[...])
        t = _elu(jnp.dot(t, w2_ref[...],
                         preferred_element_type=jnp.float32) + b2_ref[...])
        o_ref[...] = jnp.dot(t, w3_ref[...],
                             preferred_element_type=jnp.float32) + b3_ref[...]


def _reg(h, br, ws):
    (w0, b0), (w1, b1), (w2, b2), (w3, b3) = ws
    return pl.pallas_call(
        _reg_body,
        grid=(N // RP,),
        in_specs=[pl.BlockSpec((RP, HID), lambda i: (i, 0)),
                  pl.BlockSpec((RP, 1), lambda i: (i, 0)),
                  pl.BlockSpec((HID, HID), lambda i: (0, 0)),
                  pl.BlockSpec((1, HID), lambda i: (0, 0)),
                  pl.BlockSpec((HID, HID), lambda i: (0, 0)),
                  pl.BlockSpec((1, HID), lambda i: (0, 0)),
                  pl.BlockSpec((HID, HID), lambda i: (0, 0)),
                  pl.BlockSpec((1, HID), lambda i: (0, 0)),
                  pl.BlockSpec((HID, NCOMP), lambda i: (0, 0)),
                  pl.BlockSpec((1, NCOMP), lambda i: (0, 0))],
        out_specs=pl.BlockSpec((NSEG, NCOMP), lambda i: (0, 0)),
        out_shape=jax.ShapeDtypeStruct((NSEG, NCOMP), jnp.float32),
        scratch_shapes=[pltpu.VMEM((NSEG, HID), jnp.float32)],
    )(h, br, w0, b0, w1, b1, w2, b2, w3, b3)


def kernel(x, batch, params):
    xp = jnp.pad(x, ((0, 0), (0, 5)))
    wf = jnp.pad(params['ffm'][0], ((0, 5), (0, 0)))
    bf = params['ffm'][1].reshape(1, HID)
    h = _ffm(xp, wf, bf)

    batch = batch.astype(jnp.int32)
    bf32 = batch.astype(jnp.float32)
    br_rows = bf32.reshape(N, 1)

    seg_ids = jnp.arange(NSEG, dtype=batch.dtype)
    seg_start = jnp.searchsorted(batch, seg_ids, side='left')
    seg_end = jnp.searchsorted(batch, seg_ids, side='right')
    b_lo = batch[::RT]
    b_hi = batch[RT - 1::RT]
    c0 = (seg_start[b_lo] // CH).astype(jnp.int32)
    c1 = ((seg_end[b_hi] + CH - 1) // CH).astype(jnp.int32)
    bounds = jnp.stack([c0, c1], axis=1)

    gather = _make_gather()

    for lp in params['layers']:
        a, bmat, sq = _pre(h, lp['mlp1'][0], lp['mlp1'][1].reshape(1, HID))
        idx = _topk_fixed(h, sq, bf32, bounds)
        idxf = idx[:, :KNN, :].reshape(N * KNN)
        bg = gather(bmat, idxf)
        h = _edge(a, bg, h,
                  lp['mlp2'][0], lp['mlp2'][1].reshape(1, HID),
                  lp['mlp3'][0], lp['mlp3'][1].reshape(1, HID),
                  lp['ln_g'].reshape(1, HID), lp['ln_b'].reshape(1, HID),
                  lp['lin'][0], lp['lin'][1].reshape(1, HID),
                  lp['rc'].reshape(1, 1))

    ws = [(params['reg'][j][0],
           params['reg'][j][1].reshape(1, -1)) for j in range(4)]
    return _reg(h, br_rows, ws)


# speedup vs baseline: 1.1425x; 1.0042x over previous
import functools

import jax
import jax.numpy as jnp
from jax import lax
from jax.experimental import pallas as pl
from jax.experimental.pallas import tpu as pltpu
from jax.experimental.pallas import tpu_sc as plsc

N = 8192
HID = 64
KNN = 20
NSEG = 8
NCOMP = 512
CH = 256
NCHUNK = N // CH
RT = 128
RE = 128
RP = 1024
KPAD = 32
NEG_BIG = -3.0e38


def _elu(v):
    return jnp.where(v > 0, v, jnp.exp(jnp.minimum(v, 0.0)) - 1.0)


def _ffm_body(x_ref, w_ref, b_ref, o_ref):
    o_ref[...] = jnp.dot(x_ref[...], w_ref[...],
                         preferred_element_type=jnp.float32) + b_ref[...]


def _ffm(xp, wp, b):
    return pl.pallas_call(
        _ffm_body,
        grid=(N // RP,),
        in_specs=[pl.BlockSpec((RP, 8), lambda i: (i, 0)),
                  pl.BlockSpec((8, HID), lambda i: (0, 0)),
                  pl.BlockSpec((1, HID), lambda i: (0, 0))],
        out_specs=pl.BlockSpec((RP, HID), lambda i: (i, 0)),
        out_shape=jax.ShapeDtypeStruct((N, HID), jnp.float32),
    )(xp, wp, b)


def _pre_body(h_ref, w1_ref, b1_ref, a_ref, bmat_ref, sq_ref):
    h = h_ref[...]
    w1a = w1_ref[:HID, :]
    w1b = w1_ref[HID:, :]
    a_ref[...] = jnp.dot(h, w1a - w1b,
                         preferred_element_type=jnp.float32) + b1_ref[...]
    bmat_ref[:, :HID] = jnp.dot(h, w1b, preferred_element_type=jnp.float32)
    sq_ref[...] = jnp.sum(h * h, axis=1, keepdims=True)


def _pre(h, w1, b1):
    return pl.pallas_call(
        _pre_body,
        grid=(N // RP,),
        in_specs=[pl.BlockSpec((RP, HID), lambda i: (i, 0)),
                  pl.BlockSpec((2 * HID, HID), lambda i: (0, 0)),
                  pl.BlockSpec((1, HID), lambda i: (0, 0))],
        out_specs=[pl.BlockSpec((RP, HID), lambda i: (i, 0)),
                   pl.BlockSpec((RP, 2 * HID), lambda i: (i, 0)),
                   pl.BlockSpec((RP, 1), lambda i: (i, 0))],
        out_shape=[jax.ShapeDtypeStruct((N, HID), jnp.float32),
                   jax.ShapeDtypeStruct((N, 2 * HID), jnp.float32),
                   jax.ShapeDtypeStruct((N, 1), jnp.float32)],
    )(h, w1, b1)


def _topk_body(bounds_ref, hr_ref, hall_ref, sqr_ref, sqc_ref, br_ref,
               bc_ref, idx_ref, d_ref):
    i = pl.program_id(0)
    c0 = bounds_ref[i, 0]
    c1 = bounds_ref[i, 1]
    hr = hr_ref[0]
    sqr = sqr_ref[0]
    br = br_ref[0]

    col = lax.broadcasted_iota(jnp.int32, (CH, RT), 0)
    big = jnp.int32(2 ** 30)
    inf1 = jnp.full((1, RT), jnp.inf, jnp.float32)
    big1 = jnp.full((1, RT), big, jnp.int32)

    def lexlt(a, ia, b, ib):
        return (a < b) | ((a == b) & (ia < ib))

    def extract2(v, c, carry):
        m1, i1, m2, i2 = carry
        mc1 = jnp.min(v, axis=0, keepdims=True)
        il1 = jnp.argmin(v, axis=0).astype(jnp.int32).reshape(1, RT)
        ic1 = il1 + c * CH
        vx = jnp.where(col == il1, jnp.inf, v)
        mc2 = jnp.min(vx, axis=0, keepdims=True)
        ic2 = jnp.argmin(vx, axis=0).astype(jnp.int32).reshape(
            1, RT) + c * CH
        cw = lexlt(mc1, ic1, m1, i1)
        nm1 = jnp.where(cw, mc1, m1)
        ni1 = jnp.where(cw, ic1, i1)
        lm = jnp.where(cw, m1, mc1)
        li = jnp.where(cw, i1, ic1)
        ws = jnp.where(cw, mc2, m2)
        wi = jnp.where(cw, ic2, i2)
        sw = lexlt(lm, li, ws, wi)
        nm2 = jnp.where(sw, lm, ws)
        ni2 = jnp.where(sw, li, wi)
        return nm1, ni1, nm2, ni2

    def fill(c, carry):
        hc = hall_ref[c]
        dot = lax.dot_general(hc, hr, (((1,), (1,)), ((), ())),
                              preferred_element_type=jnp.float32)
        d = (sqr - 2.0 * dot) + sqc_ref[c]
        d = jnp.where(br != bc_ref[c], jnp.inf, d)
        d_ref[c] = d
        return extract2(d, c, carry)

    _, i1, _, i2 = lax.fori_loop(c0, c1, fill, (inf1, big1, inf1, big1))
    picks = [jnp.clip(i1, 0, N - 1), jnp.clip(i2, 0, N - 1)]
    prev1, prev2 = i1, i2
    npass = KNN // 2
    for p in range(1, npass):
        def scan(c, carry):
            v = d_ref[c]
            colg = col + c * CH
            v = jnp.where((colg == prev1) | (colg == prev2), jnp.inf, v)
            if p < npass - 1:
                d_ref[c] = v
            return extract2(v, c, carry)

        _, i1, _, i2 = lax.fori_loop(c0, c1, scan,
                                     (inf1, big1, inf1, big1))
        picks.append(jnp.clip(i1, 0, N - 1))
        picks.append(jnp.clip(i2, 0, N - 1))
        prev1 = i1
        prev2 = i2
    pad = [picks[-1]] * (KPAD - len(picks))
    idx_ref[0] = jnp.concatenate(picks + pad, axis=0)


def _topk(h3, sqrT, sqc, brT, bc, bounds):
    nb = N // RT
    grid_spec = pltpu.PrefetchScalarGridSpec(
        num_scalar_prefetch=1,
        grid=(nb,),
        in_specs=[pl.BlockSpec((1, RT, HID), lambda i, s: (i, 0, 0)),
                  pl.BlockSpec((NCHUNK, CH, HID), lambda i, s: (0, 0, 0)),
                  pl.BlockSpec((1, 1, RT), lambda i, s: (i, 0, 0)),
                  pl.BlockSpec((NCHUNK, CH, 1), lambda i, s: (0, 0, 0)),
                  pl.BlockSpec((1, 1, RT), lambda i, s: (i, 0, 0)),
                  pl.BlockSpec((NCHUNK, CH, 1), lambda i, s: (0, 0, 0))],
        out_specs=pl.BlockSpec((1, KPAD, RT), lambda i, s: (i, 0, 0)),
        scratch_shapes=[pltpu.VMEM((NCHUNK, CH, RT), jnp.float32)],
    )
    hrows = h3.reshape(nb, RT, HID)
    return pl.pallas_call(
        _topk_body,
        grid_spec=grid_spec,
        out_shape=jax.ShapeDtypeStruct((nb, KPAD, RT), jnp.int32),
    )(bounds, hrows, h3, sqrT, sqc, brT, bc)


def _topk_fixed(h, sq, bf32, bounds):
    nb = N // RT
    return _topk(h.reshape(NCHUNK, CH, HID),
                 sq.reshape(nb, 1, RT),
                 sq.reshape(NCHUNK, CH, 1),
                 bf32.reshape(nb, 1, RT),
                 bf32.reshape(NCHUNK, CH, 1),
                 bounds)


def _make_gather():
    info = plsc.get_sparse_core_info()
    nw = info.num_cores * info.num_subcores
    total = N * KNN
    bpw = total // nw
    chunk = 320
    nchunks = bpw // chunk
    mesh = plsc.VectorSubcoreMesh(core_axis_name="c", subcore_axis_name="s")

    @functools.partial(
        pl.kernel, mesh=mesh,
        out_type=jax.ShapeDtypeStruct((total, 2 * HID), jnp.float32),
        scratch_types=[pltpu.VMEM((bpw,), jnp.int32),
                       pltpu.VMEM((chunk, 2 * HID), jnp.float32),
                       pltpu.VMEM((chunk, 2 * HID), jnp.float32),
                       pltpu.SemaphoreType.DMA,
                       pltpu.SemaphoreType.DMA],
    )
    def gather(table_hbm, idx_hbm, out_hbm, idx_v, rows0, rows1, sem0,
               sem1):
        wid = lax.axis_index("s") * info.num_cores + lax.axis_index("c")
        base = wid * bpw
        pltpu.sync_copy(idx_hbm.at[pl.ds(base, bpw)], idx_v)
        bufs = [rows0, rows1]
        sems = [sem0, sem1]

        def start(t):
            idx_slice = idx_v.at[pl.ds(t * chunk, chunk)]
            return pltpu.async_copy(table_hbm.at[idx_slice],
                                    bufs[t % 2], sems[t % 2])

        pending = start(0)
        for t in range(nchunks):
            pending.wait()
            if t + 1 < nchunks:
                pending = start(t + 1)
            pltpu.sync_copy(bufs[t % 2],
                            out_hbm.at[pl.ds(base + t * chunk, chunk)])

    return gather


def _edge_body(a_ref, bg_ref, h_ref, w2_ref, b2_ref, w3_ref, b3_ref,
               g_ref, bl_ref, wl_ref, bll_ref, rc_ref, o_ref):
    a = a_ref[...]
    ae = jnp.broadcast_to(a[None, :, :], (KNN, RE, HID)).reshape(
        KNN * RE, HID)
    m1 = _elu(ae + bg_ref[:, :HID])
    m2 = _elu(jnp.dot(m1, w2_ref[...],
                      preferred_element_type=jnp.float32) + b2_ref[...])
    m3 = jnp.dot(m2, w3_ref[...],
                 preferred_element_type=jnp.float32) + b3_ref[...]
    agg = jnp.max(m3.reshape(KNN, RE, HID), axis=0)
    mu = jnp.mean(agg, axis=1, keepdims=True)
    var = jnp.mean((agg - mu) ** 2, axis=1, keepdims=True)
    y = (agg - mu) / jnp.sqrt(var + 1e-5) * g_ref[...] + bl_ref[...]
    o = jnp.dot(_elu(y), wl_ref[...],
                preferred_element_type=jnp.float32) + bll_ref[...]
    o_ref[...] = h_ref[...] + rc_ref[0, 0] * o


def _edge(a, bg, h, w2, b2, w3, b3, g, bln, wl, bl, rc):
    return pl.pallas_call(
        _edge_body,
        grid=(N // RE,),
        in_specs=[pl.BlockSpec((RE, HID), lambda i: (i, 0)),
                  pl.BlockSpec((RE * KNN, 2 * HID), lambda i: (i, 0)),
                  pl.BlockSpec((RE, HID), lambda i: (i, 0)),
                  pl.BlockSpec((HID, HID), lambda i: (0, 0)),
                  pl.BlockSpec((1, HID), lambda i: (0, 0)),
                  pl.BlockSpec((HID, HID), lambda i: (0, 0)),
                  pl.BlockSpec((1, HID), lambda i: (0, 0)),
                  pl.BlockSpec((1, HID), lambda i: (0, 0)),
                  pl.BlockSpec((1, HID), lambda i: (0, 0)),
                  pl.BlockSpec((HID, HID), lambda i: (0, 0)),
                  pl.BlockSpec((1, HID), lambda i: (0, 0)),
                  pl.BlockSpec((1, 1), lambda i: (0, 0))],
        out_specs=pl.BlockSpec((RE, HID), lambda i: (i, 0)),
        out_shape=jax.ShapeDtypeStruct((N, HID), jnp.float32),
    )(a, bg, h, w2, b2, w3, b3, g, bln, wl, bl, rc)


def _reg_body(h_ref, br_ref, w0_ref, b0_ref, w1_ref, b1_ref, w2_ref,
              b2_ref, w3_ref, b3_ref, o_ref, pool_ref):
    i = pl.program_id(0)
    nb = pl.num_programs(0)

    @pl.when(i == 0)
    def _():
        pool_ref[...] = jnp.full((NSEG, HID), NEG_BIG, jnp.float32)

    r = jnp.dot(h_ref[...], w0_ref[...],
                preferred_element_type=jnp.float32) + b0_ref[...]
    br = br_ref[...]
    for s in range(NSEG):
        rs = jnp.where(br == jnp.float32(s), r, NEG_BIG)
        ps = jnp.max(rs, axis=0, keepdims=True)
        pool_ref[pl.ds(s, 1), :] = jnp.maximum(pool_ref[pl.ds(s, 1), :], ps)

    @pl.when(i == nb - 1)
    def _():
        p = pool_ref[...]
        t = _elu(jnp.dot(p, w1_ref[...],
                         preferred_element_type=jnp.float32) + b1_ref[...])
        t = _elu(jnp.dot(t, w2_ref[...],
                         preferred_element_type=jnp.float32) + b2_ref[...])
        o_ref[...] = jnp.dot(t, w3_ref[...],
                             preferred_element_type=jnp.float32) + b3_ref[...]


def _reg(h, br, ws):
    (w0, b0), (w1, b1), (w2, b2), (w3, b3) = ws
    return pl.pallas_call(
        _reg_body,
        grid=(N // RP,),
        in_specs=[pl.BlockSpec((RP, HID), lambda i: (i, 0)),
                  pl.BlockSpec((RP, 1), lambda i: (i, 0)),
                  pl.BlockSpec((HID, HID), lambda i: (0, 0)),
                  pl.BlockSpec((1, HID), lambda i: (0, 0)),
                  pl.BlockSpec((HID, HID), lambda i: (0, 0)),
                  pl.BlockSpec((1, HID), lambda i: (0, 0)),
                  pl.BlockSpec((HID, HID), lambda i: (0, 0)),
                  pl.BlockSpec((1, HID), lambda i: (0, 0)),
                  pl.BlockSpec((HID, NCOMP), lambda i: (0, 0)),
                  pl.BlockSpec((1, NCOMP), lambda i: (0, 0))],
        out_specs=pl.BlockSpec((NSEG, NCOMP), lambda i: (0, 0)),
        out_shape=jax.ShapeDtypeStruct((NSEG, NCOMP), jnp.float32),
        scratch_shapes=[pltpu.VMEM((NSEG, HID), jnp.float32)],
    )(h, br, w0, b0, w1, b1, w2, b2, w3, b3)


def kernel(x, batch, params):
    xp = jnp.pad(x, ((0, 0), (0, 5)))
    wf = jnp.pad(params['ffm'][0], ((0, 5), (0, 0)))
    bf = params['ffm'][1].reshape(1, HID)
    h = _ffm(xp, wf, bf)

    batch = batch.astype(jnp.int32)
    bf32 = batch.astype(jnp.float32)
    br_rows = bf32.reshape(N, 1)

    seg_ids = jnp.arange(NSEG, dtype=batch.dtype)
    seg_start = jnp.searchsorted(batch, seg_ids, side='left')
    seg_end = jnp.searchsorted(batch, seg_ids, side='right')
    b_lo = batch[::RT]
    b_hi = batch[RT - 1::RT]
    c0 = (seg_start[b_lo] // CH).astype(jnp.int32)
    c1 = ((seg_end[b_hi] + CH - 1) // CH).astype(jnp.int32)
    bounds = jnp.stack([c0, c1], axis=1)

    gather = _make_gather()

    for lp in params['layers']:
        a, bmat, sq = _pre(h, lp['mlp1'][0], lp['mlp1'][1].reshape(1, HID))
        idx = _topk_fixed(h, sq, bf32, bounds)
        idxf = idx[:, :KNN, :].reshape(N * KNN)
        bg = gather(bmat, idxf)
        h = _edge(a, bg, h,
                  lp['mlp2'][0], lp['mlp2'][1].reshape(1, HID),
                  lp['mlp3'][0], lp['mlp3'][1].reshape(1, HID),
                  lp['ln_g'].reshape(1, HID), lp['ln_b'].reshape(1, HID),
                  lp['lin'][0], lp['lin'][1].reshape(1, HID),
                  lp['rc'].reshape(1, 1))

    ws = [(params['reg'][j][0],
           params['reg'][j][1].reshape(1, -1)) for j in range(4)]
    return _reg(h, br_rows, ws)
